# Initial kernel scaffold; baseline (speedup 1.0000x reference)
import jax
import jax.numpy as jnp
from jax import lax
from jax.experimental import pallas as pl
from jax.experimental.pallas import tpu as pltpu
from jax.experimental.pallas import tpu_sc as plsc

N = 10000
E = 320000
D = 128
DE = 16
G = 128
C = 10
BN_INV = 1.0 / (1.0 + 1e-5) ** 0.5

_NC = 2
_NS = 16
_TILES = _NC * _NS
_EPT = E // _TILES
_CH = 80
_NCH = _EPT // _CH
_RPS = N // _NS
_ZR = 125

_BE = 4000
_BN = 2000


def _ee_body(ea_ref, w1_ref, b1_ref, w2_ref, b2_ref, g_ref, bt_ref, out_ref):
    ea = ea_ref[...]
    t = lax.dot_general(ea, w1_ref[...], (((1,), (0,)), ((), ())),
                        preferred_element_type=jnp.float32)
    t = jnp.maximum(t + b1_ref[...], 0.0)
    t = lax.dot_general(t, w2_ref[...], (((1,), (0,)), ((), ())),
                        preferred_element_type=jnp.float32)
    t = jnp.maximum(t + b2_ref[...], 0.0)
    out_ref[...] = t * g_ref[...] + bt_ref[...]


_ee_call = pl.pallas_call(
    _ee_body,
    grid=(E // _BE,),
    in_specs=[
        pl.BlockSpec((_BE, DE), lambda i: (i, 0)),
        pl.BlockSpec((DE, D), lambda i: (0, 0)),
        pl.BlockSpec((1, D), lambda i: (0, 0)),
        pl.BlockSpec((D, D), lambda i: (0, 0)),
        pl.BlockSpec((1, D), lambda i: (0, 0)),
        pl.BlockSpec((1, D), lambda i: (0, 0)),
        pl.BlockSpec((1, D), lambda i: (0, 0)),
    ],
    out_specs=pl.BlockSpec((_BE, D), lambda i: (i, 0)),
    out_shape=jax.ShapeDtypeStruct((E, D), jnp.float32),
)


def _sc_body(h_hbm, e_hbm, src_hbm, dst_hbm, out_hbm,
             sidx, didx, hbuf, ebuf, zbuf, agg_sh, sem):
    cid = lax.axis_index("c")
    sid = lax.axis_index("s")
    wid = cid * _NS + sid

    def _zrow(k, carry):
        for d2 in range(D // 16):
            zbuf[k, pl.ds(d2 * 16, 16)] = jnp.zeros((16,), jnp.float32)
        return carry
    lax.fori_loop(0, _ZR, _zrow, 0)
    row0 = sid * _RPS
    for j in range(_RPS // _ZR):
        pltpu.sync_copy(zbuf, agg_sh.at[pl.ds(row0 + j * _ZR, _ZR)])

    pltpu.sync_copy(src_hbm.at[wid], sidx)
    pltpu.sync_copy(dst_hbm.at[wid], didx)
    plsc.subcore_barrier()

    base = wid * _EPT

    def _chunk(c, carry):
        pltpu.sync_copy(e_hbm.at[pl.ds(base + c * _CH, _CH)], ebuf)
        pltpu.async_copy(h_hbm.at[sidx.at[c]], hbuf, sem).wait()

        def _row(k, inner):
            for d2 in range(D // 16):
                sl = pl.ds(d2 * 16, 16)
                ebuf[k, sl] = jnp.maximum(ebuf[k, sl] + hbuf[k, sl], 0.0)
            return inner
        lax.fori_loop(0, _CH, _row, 0)
        pltpu.sync_copy(ebuf, agg_sh.at[didx.at[c]], add=True)
        return carry
    lax.fori_loop(0, _NCH, _chunk, 0)
    plsc.subcore_barrier()

    for j in range(_RPS // _ZR):
        r = row0 + j * _ZR
        pltpu.sync_copy(agg_sh.at[pl.ds(r, _ZR)], zbuf)
        pltpu.sync_copy(zbuf, out_hbm.at[pl.ds(cid * N + r, _ZR)])


_sc_call = pl.kernel(
    _sc_body,
    out_type=jax.ShapeDtypeStruct((_NC * N, D), jnp.float32),
    mesh=plsc.VectorSubcoreMesh(core_axis_name="c", subcore_axis_name="s"),
    scratch_types=[
        pltpu.VMEM((_NCH, _CH), jnp.int32),
        pltpu.VMEM((_NCH, _CH), jnp.int32),
        pltpu.VMEM((_CH, D), jnp.float32),
        pltpu.VMEM((_CH, D), jnp.float32),
        pltpu.VMEM((_ZR, D), jnp.float32),
        pltpu.VMEM_SHARED((N, D), jnp.float32),
        pltpu.SemaphoreType.DMA,
    ],
)


def _mlp_body(h_ref, agg_ref, w1_ref, b1_ref, w2_ref, b2_ref, g_ref, bt_ref,
              out_ref):
    t = h_ref[...] + agg_ref[0] + agg_ref[1]
    t = lax.dot_general(t, w1_ref[...], (((1,), (0,)), ((), ())),
                        preferred_element_type=jnp.float32)
    t = jnp.maximum(t + b1_ref[...], 0.0)
    t = lax.dot_general(t, w2_ref[...], (((1,), (0,)), ((), ())),
                        preferred_element_type=jnp.float32)
    t = jnp.maximum(t + b2_ref[...], 0.0)
    out_ref[...] = t * g_ref[...] + bt_ref[...]


_mlp_call = pl.pallas_call(
    _mlp_body,
    grid=(N // _BN,),
    in_specs=[
        pl.BlockSpec((_BN, D), lambda i: (i, 0)),
        pl.BlockSpec((_NC, _BN, D), lambda i: (0, i, 0)),
        pl.BlockSpec((D, D), lambda i: (0, 0)),
        pl.BlockSpec((1, D), lambda i: (0, 0)),
        pl.BlockSpec((D, D), lambda i: (0, 0)),
        pl.BlockSpec((1, D), lambda i: (0, 0)),
        pl.BlockSpec((1, D), lambda i: (0, 0)),
        pl.BlockSpec((1, D), lambda i: (0, 0)),
    ],
    out_specs=pl.BlockSpec((_BN, D), lambda i: (i, 0)),
    out_shape=jax.ShapeDtypeStruct((N, D), jnp.float32),
)


def _pool_body(b_ref, h_ref, l1w_ref, l1b_ref, l2w_ref, l2b_ref, out_ref,
               acc_ref, cnt_ref):
    i = pl.program_id(0)

    @pl.when(i == 0)
    def _init():
        acc_ref[...] = jnp.zeros_like(acc_ref)
        cnt_ref[...] = jnp.zeros_like(cnt_ref)

    onehot = (b_ref[...] == lax.broadcasted_iota(jnp.int32, (_BN, G), 1)
              ).astype(jnp.float32)
    acc_ref[...] += lax.dot_general(onehot, h_ref[...], (((0,), (0,)), ((), ())),
                                    preferred_element_type=jnp.float32)
    cnt_ref[...] += lax.dot_general(onehot, jnp.ones((_BN, 1), jnp.float32),
                                    (((0,), (0,)), ((), ())),
                                    preferred_element_type=jnp.float32)

    @pl.when(i == pl.num_programs(0) - 1)
    def _fin():
        pooled = acc_ref[...] / jnp.maximum(cnt_ref[...], 1.0)
        o = lax.dot_general(pooled, l1w_ref[...], (((1,), (0,)), ((), ())),
                            preferred_element_type=jnp.float32)
        o = jnp.maximum(o + l1b_ref[...], 0.0)
        logits = lax.dot_general(o, l2w_ref[...], (((1,), (0,)), ((), ())),
                                 preferred_element_type=jnp.float32)
        logits = logits + l2b_ref[...]
        m = jnp.max(logits, axis=1, keepdims=True)
        lse = m + jnp.log(jnp.sum(jnp.exp(logits - m), axis=1, keepdims=True))
        out_ref[...] = logits - lse


_pool_call = pl.pallas_call(
    _pool_body,
    grid=(N // _BN,),
    in_specs=[
        pl.BlockSpec((_BN, 1), lambda i: (i, 0)),
        pl.BlockSpec((_BN, D), lambda i: (i, 0)),
        pl.BlockSpec((D, D), lambda i: (0, 0)),
        pl.BlockSpec((1, D), lambda i: (0, 0)),
        pl.BlockSpec((D, C), lambda i: (0, 0)),
        pl.BlockSpec((1, C), lambda i: (0, 0)),
    ],
    out_specs=pl.BlockSpec((G, C), lambda i: (0, 0)),
    out_shape=jax.ShapeDtypeStruct((G, C), jnp.float32),
    scratch_shapes=[
        pltpu.VMEM((G, D), jnp.float32),
        pltpu.VMEM((G, 1), jnp.float32),
    ],
)


def kernel(x, edge_index, edge_attr, batch,
           eW1, eb1, eW2, eb2, eg, ebt,
           mW1, mb1, mW2, mb2, mg, mbt,
           lin1_W, lin1_b, lin2_W, lin2_b):
    src = edge_index[0].astype(jnp.int32).reshape(_TILES, _NCH, _CH)
    dst = edge_index[1].astype(jnp.int32).reshape(_TILES, _NCH, _CH)
    batch2 = batch.astype(jnp.int32).reshape(N, 1)
    eg_s = (eg * BN_INV).reshape(3, 1, D)
    ebt2 = ebt.reshape(3, 1, D)
    mg_s = (mg * BN_INV).reshape(3, 1, D)
    mbt2 = mbt.reshape(3, 1, D)
    eb1_2 = eb1.reshape(3, 1, D)
    eb2_2 = eb2.reshape(3, 1, D)
    mb1_2 = mb1.reshape(3, 1, D)
    mb2_2 = mb2.reshape(3, 1, D)

    h = x
    for l in range(3):
        e = _ee_call(edge_attr, eW1[l], eb1_2[l], eW2[l], eb2_2[l],
                     eg_s[l], ebt2[l])
        parts = _sc_call(h, e, src, dst)
        h = _mlp_call(h, parts.reshape(_NC, N, D), mW1[l], mb1_2[l],
                      mW2[l], mb2_2[l], mg_s[l], mbt2[l])
    return _pool_call(batch2, h, lin1_W, lin1_b.reshape(1, D),
                      lin2_W, lin2_b.reshape(1, C))


# speedup vs baseline: 2.5694x; 2.5694x over previous
import jax
import jax.numpy as jnp
from jax import lax
from jax.experimental import pallas as pl
from jax.experimental.pallas import tpu as pltpu
from jax.experimental.pallas import tpu_sc as plsc

N = 10000
E = 320000
D = 128
DE = 16
G = 128
C = 10
BN_INV = 1.0 / (1.0 + 1e-5) ** 0.5

_NC = 2
_NS = 16
_TILES = _NC * _NS
_EPT = E // _TILES
_CH = 80
_NCH = _EPT // _CH
_NP = 10240
_RPS = _NP // _NS
_ZR = 128

_BE = 4000
_BN = 2000


def _ee_body(ea_ref, w1_ref, b1_ref, w2_ref, b2_ref, g_ref, bt_ref, out_ref):
    ea = ea_ref[...]
    t = lax.dot_general(ea, w1_ref[...], (((1,), (0,)), ((), ())),
                        preferred_element_type=jnp.float32)
    t = jnp.maximum(t + b1_ref[...], 0.0)
    t = lax.dot_general(t, w2_ref[...], (((1,), (0,)), ((), ())),
                        preferred_element_type=jnp.float32)
    t = jnp.maximum(t + b2_ref[...], 0.0)
    out_ref[...] = t * g_ref[...] + bt_ref[...]


_ee_call = pl.pallas_call(
    _ee_body,
    grid=(E // _BE,),
    in_specs=[
        pl.BlockSpec((_BE, DE), lambda i: (i, 0)),
        pl.BlockSpec((DE, D), lambda i: (0, 0)),
        pl.BlockSpec((1, D), lambda i: (0, 0)),
        pl.BlockSpec((D, D), lambda i: (0, 0)),
        pl.BlockSpec((1, D), lambda i: (0, 0)),
        pl.BlockSpec((1, D), lambda i: (0, 0)),
        pl.BlockSpec((1, D), lambda i: (0, 0)),
    ],
    out_specs=pl.BlockSpec((_BE, D), lambda i: (i, 0)),
    out_shape=jax.ShapeDtypeStruct((E, D), jnp.float32),
)


def _sc_body(h_hbm, e_hbm, src_hbm, dst_hbm, out_hbm,
             sidx, didx, hbuf, ebuf, zbuf, agg_sh, sem):
    cid = lax.axis_index("c")
    sid = lax.axis_index("s")
    wid = cid * _NS + sid

    def _zrow(k, carry):
        for d2 in range(D // 16):
            zbuf[k, pl.ds(d2 * 16, 16)] = jnp.zeros((16,), jnp.float32)
        return carry
    lax.fori_loop(0, _ZR, _zrow, 0)
    row0 = sid * _RPS
    for j in range(_RPS // _ZR):
        pltpu.sync_copy(zbuf, agg_sh.at[pl.ds(row0 + j * _ZR, _ZR)])

    plsc.subcore_barrier()

    base = wid * _EPT

    def _chunk(c, carry):
        eb = base + c * _CH
        pltpu.sync_copy(src_hbm.at[pl.ds(eb, _CH)], sidx)
        pltpu.sync_copy(dst_hbm.at[pl.ds(eb, _CH)], didx)
        pltpu.sync_copy(e_hbm.at[pl.ds(eb, _CH)], ebuf)
        pltpu.async_copy(h_hbm.at[sidx], hbuf, sem).wait()

        def _row(k, inner):
            for d2 in range(D // 16):
                sl = pl.ds(d2 * 16, 16)
                ebuf[k, sl] = jnp.maximum(ebuf[k, sl] + hbuf[k, sl], 0.0)
            return inner
        lax.fori_loop(0, _CH, _row, 0)
        pltpu.sync_copy(ebuf, agg_sh.at[didx], add=True)
        return carry
    lax.fori_loop(0, _NCH, _chunk, 0)
    plsc.subcore_barrier()

    for j in range(_RPS // _ZR):
        r = row0 + j * _ZR
        pltpu.sync_copy(agg_sh.at[pl.ds(r, _ZR)], zbuf)
        pltpu.sync_copy(zbuf, out_hbm.at[pl.ds(cid * _NP + r, _ZR)])


_sc_call = pl.kernel(
    _sc_body,
    out_type=jax.ShapeDtypeStruct((_NC * _NP, D), jnp.float32),
    mesh=plsc.VectorSubcoreMesh(core_axis_name="c", subcore_axis_name="s",
                                num_cores=_NC, num_subcores=_NS),
    scratch_types=[
        pltpu.VMEM((_CH,), jnp.int32),
        pltpu.VMEM((_CH,), jnp.int32),
        pltpu.VMEM((_CH, D), jnp.float32),
        pltpu.VMEM((_CH, D), jnp.float32),
        pltpu.VMEM((_ZR, D), jnp.float32),
        pltpu.VMEM_SHARED((_NP, D), jnp.float32),
        pltpu.SemaphoreType.DMA,
    ],
)


def _mlp_body(h_ref, agg_ref, w1_ref, b1_ref, w2_ref, b2_ref, g_ref, bt_ref,
              out_ref):
    t = h_ref[...] + agg_ref[0] + agg_ref[1]
    t = lax.dot_general(t, w1_ref[...], (((1,), (0,)), ((), ())),
                        preferred_element_type=jnp.float32)
    t = jnp.maximum(t + b1_ref[...], 0.0)
    t = lax.dot_general(t, w2_ref[...], (((1,), (0,)), ((), ())),
                        preferred_element_type=jnp.float32)
    t = jnp.maximum(t + b2_ref[...], 0.0)
    out_ref[...] = t * g_ref[...] + bt_ref[...]


_mlp_call = pl.pallas_call(
    _mlp_body,
    grid=(N // _BN,),
    in_specs=[
        pl.BlockSpec((_BN, D), lambda i: (i, 0)),
        pl.BlockSpec((_NC, _BN, D), lambda i: (0, i, 0)),
        pl.BlockSpec((D, D), lambda i: (0, 0)),
        pl.BlockSpec((1, D), lambda i: (0, 0)),
        pl.BlockSpec((D, D), lambda i: (0, 0)),
        pl.BlockSpec((1, D), lambda i: (0, 0)),
        pl.BlockSpec((1, D), lambda i: (0, 0)),
        pl.BlockSpec((1, D), lambda i: (0, 0)),
    ],
    out_specs=pl.BlockSpec((_BN, D), lambda i: (i, 0)),
    out_shape=jax.ShapeDtypeStruct((N, D), jnp.float32),
)


def _pool_body(b_ref, h_ref, l1w_ref, l1b_ref, l2w_ref, l2b_ref, out_ref,
               acc_ref, cnt_ref):
    i = pl.program_id(0)

    @pl.when(i == 0)
    def _init():
        acc_ref[...] = jnp.zeros_like(acc_ref)
        cnt_ref[...] = jnp.zeros_like(cnt_ref)

    onehot = (b_ref[...] == lax.broadcasted_iota(jnp.int32, (_BN, G), 1)
              ).astype(jnp.float32)
    acc_ref[...] += lax.dot_general(onehot, h_ref[...], (((0,), (0,)), ((), ())),
                                    preferred_element_type=jnp.float32)
    cnt_ref[...] += lax.dot_general(onehot, jnp.ones((_BN, 1), jnp.float32),
                                    (((0,), (0,)), ((), ())),
                                    preferred_element_type=jnp.float32)

    @pl.when(i == pl.num_programs(0) - 1)
    def _fin():
        pooled = acc_ref[...] / jnp.maximum(cnt_ref[...], 1.0)
        o = lax.dot_general(pooled, l1w_ref[...], (((1,), (0,)), ((), ())),
                            preferred_element_type=jnp.float32)
        o = jnp.maximum(o + l1b_ref[...], 0.0)
        logits = lax.dot_general(o, l2w_ref[...], (((1,), (0,)), ((), ())),
                                 preferred_element_type=jnp.float32)
        logits = logits + l2b_ref[...]
        m = jnp.max(logits, axis=1, keepdims=True)
        lse = m + jnp.log(jnp.sum(jnp.exp(logits - m), axis=1, keepdims=True))
        out_ref[...] = logits - lse


_pool_call = pl.pallas_call(
    _pool_body,
    grid=(N // _BN,),
    in_specs=[
        pl.BlockSpec((_BN, 1), lambda i: (i, 0)),
        pl.BlockSpec((_BN, D), lambda i: (i, 0)),
        pl.BlockSpec((D, D), lambda i: (0, 0)),
        pl.BlockSpec((1, D), lambda i: (0, 0)),
        pl.BlockSpec((D, C), lambda i: (0, 0)),
        pl.BlockSpec((1, C), lambda i: (0, 0)),
    ],
    out_specs=pl.BlockSpec((G, C), lambda i: (0, 0)),
    out_shape=jax.ShapeDtypeStruct((G, C), jnp.float32),
    scratch_shapes=[
        pltpu.VMEM((G, D), jnp.float32),
        pltpu.VMEM((G, 1), jnp.float32),
    ],
)


def kernel(x, edge_index, edge_attr, batch,
           eW1, eb1, eW2, eb2, eg, ebt,
           mW1, mb1, mW2, mb2, mg, mbt,
           lin1_W, lin1_b, lin2_W, lin2_b):
    src = edge_index[0].astype(jnp.int32)
    dst = edge_index[1].astype(jnp.int32)
    batch2 = batch.astype(jnp.int32).reshape(N, 1)
    eg_s = (eg * BN_INV).reshape(3, 1, D)
    ebt2 = ebt.reshape(3, 1, D)
    mg_s = (mg * BN_INV).reshape(3, 1, D)
    mbt2 = mbt.reshape(3, 1, D)
    eb1_2 = eb1.reshape(3, 1, D)
    eb2_2 = eb2.reshape(3, 1, D)
    mb1_2 = mb1.reshape(3, 1, D)
    mb2_2 = mb2.reshape(3, 1, D)

    h = x
    for l in range(3):
        e = _ee_call(edge_attr, eW1[l], eb1_2[l], eW2[l], eb2_2[l],
                     eg_s[l], ebt2[l])
        parts = _sc_call(h, e, src, dst)
        h = _mlp_call(h, parts.reshape(_NC, _NP, D), mW1[l], mb1_2[l],
                      mW2[l], mb2_2[l], mg_s[l], mbt2[l])
    return _pool_call(batch2, h, lin1_W, lin1_b.reshape(1, D),
                      lin2_W, lin2_b.reshape(1, C))
